# baseline (speedup 1.0000x reference)
import functools

import jax
import jax.numpy as jnp
from jax import lax
from jax.experimental import pallas as pl
from jax.experimental.pallas import tpu as pltpu
from jax.experimental.pallas import tpu_sc as plsc

_NC = 2
_NS = 16
_NW = _NC * _NS
_CW = 512


@functools.lru_cache(maxsize=None)
def _scan_call(B, D, V):
    mesh = plsc.VectorSubcoreMesh(core_axis_name="c", subcore_axis_name="s")
    v_main = (V // _CW) * _CW
    n_chunks = v_main // _CW
    tail = V - v_main
    per_tile = n_chunks // _NW
    extra = n_chunks - per_tile * _NW

    @functools.partial(
        pl.kernel,
        mesh=mesh,
        out_type=jax.ShapeDtypeStruct((B * D,), jnp.float32),
        scratch_types=[
            pltpu.VMEM((B,), jnp.int32),
            pltpu.VMEM((B + 16,), jnp.int32),
            pltpu.VMEM((B + 16,), jnp.int32),
            pltpu.VMEM((D, _CW), jnp.float32),
            pltpu.VMEM((D, _CW), jnp.float32),
            pltpu.VMEM((64, D), jnp.float32),
            pltpu.SemaphoreType.DMA,
            pltpu.SemaphoreType.DMA,
        ],
        compiler_params=pltpu.CompilerParams(needs_layout_passes=False),
    )
    def k(idx_hbm, tabT_hbm, auxT_hbm, out_hbm, idx_v, pos_v, row_v,
          cka, ckb, ring_v, csem, wsem):
        wid = lax.axis_index("s") * _NC + lax.axis_index("c")
        my_chunks = jnp.where(wid < extra, per_tile + 1, per_tile)
        c0 = wid * per_tile + jnp.minimum(wid, extra)
        lo = c0 * _CW
        hi = lo + my_chunks * _CW
        is_last = wid == _NW - 1
        hi_eff = jnp.where(is_last, V, hi)

        def fetch(ci, buf):
            pltpu.async_copy(
                tabT_hbm.at[:, pl.ds((c0 + ci) * _CW, _CW)], buf, csem
            )

        @pl.when(my_chunks > 0)
        def _():
            fetch(jnp.int32(0), cka)

        if tail:
            @pl.when(is_last)
            def _():
                pltpu.async_copy(auxT_hbm, ckb.at[:, pl.ds(0, 128)], csem)

        pltpu.sync_copy(idx_hbm, idx_v)

        def scan_body(g, n):
            iv = idx_v[pl.ds(g * 16, 16)]
            bv = lax.iota(jnp.int32, 16) + g * 16
            ge = jnp.minimum(jnp.maximum(iv - lo + 1, 0), 1)
            lt = jnp.minimum(jnp.maximum(hi_eff - iv, 0), 1)
            m01 = ge * lt
            s = plsc.cumsum(m01)
            tgt = (n + s - m01) * m01 + B * (1 - m01)
            plsc.store_scatter(pos_v, [tgt], bv)
            plsc.store_scatter(row_v, [tgt], iv)
            return n + s[15]

        n_hits = lax.fori_loop(0, B // 16, scan_body, jnp.int32(0), unroll=2)
        n_groups = (n_hits + 15) // 16

        def drain_one():
            pltpu.make_async_copy(
                idx_hbm.at[pl.ds(0, D)], row_v.at[pl.ds(0, D)], wsem
            ).wait()

        def select_and_emit(buf, chunk_lo, width, cnt0):
            riota = lax.iota(jnp.int32, 16)

            def grp(g, cnt):
                rv = row_v[pl.ds(g * 16, 16)]
                bv = pos_v[pl.ds(g * 16, 16)]
                c = cnt
                for l in range(16):
                    r_l = rv[l]
                    b_l = bv[l]
                    ok = jnp.logical_and(
                        jnp.logical_and(r_l >= chunk_lo,
                                        r_l < chunk_lo + width),
                        g * 16 + l < n_hits,
                    )

                    @pl.when(ok)
                    def _(r_l=r_l, b_l=b_l, c=c):
                        @pl.when(c >= 64)
                        def _():
                            drain_one()
                        col = jnp.full((16,), r_l - chunk_lo, jnp.int32)
                        slot = lax.rem(c, 64)
                        for cc in range(0, D, 16):
                            vals = plsc.load_gather(
                                buf, [riota + cc, col]
                            )
                            ring_v[slot, pl.ds(cc, 16)] = vals
                        pltpu.async_copy(
                            ring_v.at[slot],
                            out_hbm.at[pl.ds(b_l * D, D)],
                            wsem,
                        )
                    c = c + ok.astype(jnp.int32)
                return c

            return lax.fori_loop(0, n_groups, grp, cnt0)

        cnt0 = jnp.int32(0)
        if tail:
            def aux_path(c):
                pltpu.make_async_copy(
                    auxT_hbm, ckb.at[:, pl.ds(0, 128)], csem
                ).wait()
                return select_and_emit(ckb, jnp.int32(v_main), tail, c)

            cnt0 = lax.cond(is_last, aux_path, lambda c: c, cnt0)

        def body(ci, cnt):
            pltpu.make_async_copy(
                tabT_hbm.at[:, pl.ds(0, _CW)], cka, csem
            ).wait()
            even = lax.rem(ci, 2) == 0

            @pl.when(ci + 1 < my_chunks)
            def _():
                @pl.when(even)
                def _():
                    fetch(ci + 1, ckb)

                @pl.when(jnp.logical_not(even))
                def _():
                    fetch(ci + 1, cka)

            return lax.cond(
                even,
                lambda c: select_and_emit(cka, lo + ci * _CW, _CW, c),
                lambda c: select_and_emit(ckb, lo + ci * _CW, _CW, c),
                cnt,
            )

        total = lax.fori_loop(0, my_chunks, body, cnt0)

        def dr(i, _):
            drain_one()
            return ()

        lax.fori_loop(0, jnp.minimum(total, 64), dr, ())

    return k


def kernel(index, table):
    (B,) = index.shape
    V, D = table.shape
    assert B % 16 == 0 and D % 16 == 0
    idx = index.astype(jnp.int32)
    tabT = jnp.swapaxes(table, 0, 1)
    v_main = (V // _CW) * _CW
    tail = V - v_main
    auxT = jnp.swapaxes(
        jnp.pad(table[v_main:], ((0, 128 - tail), (0, 0))), 0, 1
    )
    out1d = _scan_call(B, D, V)(idx, tabT, auxT)
    return jnp.reshape(out1d, (B, D))


# speedup vs baseline: 2.5728x; 2.5728x over previous
import functools

import jax
import jax.numpy as jnp
from jax import lax
from jax.experimental import pallas as pl
from jax.experimental.pallas import tpu as pltpu
from jax.experimental.pallas import tpu_sc as plsc

_NC = 2
_NS = 16
_NW = _NC * _NS
_CW = 512


@functools.lru_cache(maxsize=None)
def _scan_call(B, D, V):
    mesh = plsc.VectorSubcoreMesh(core_axis_name="c", subcore_axis_name="s")
    v_main = (V // _CW) * _CW
    n_chunks = v_main // _CW
    tail = V - v_main
    per_tile = n_chunks // _NW
    extra = n_chunks - per_tile * _NW

    @functools.partial(
        pl.kernel,
        mesh=mesh,
        out_type=jax.ShapeDtypeStruct((B * D,), jnp.float32),
        scratch_types=[
            pltpu.VMEM((B,), jnp.int32),
            pltpu.VMEM((B + 16,), jnp.int32),
            pltpu.VMEM((B + 16,), jnp.int32),
            pltpu.VMEM((D, _CW), jnp.float32),
            pltpu.VMEM((D, _CW), jnp.float32),
            pltpu.VMEM((64, D), jnp.float32),
            pltpu.SemaphoreType.DMA,
            pltpu.SemaphoreType.DMA,
        ],
        compiler_params=pltpu.CompilerParams(needs_layout_passes=False),
    )
    def k(idx_hbm, tabT_hbm, auxT_hbm, out_hbm, idx_v, pos_v, row_v,
          cka, ckb, ring_v, csem, wsem):
        wid = lax.axis_index("s") * _NC + lax.axis_index("c")
        my_chunks = jnp.where(wid < extra, per_tile + 1, per_tile)
        c0 = wid * per_tile + jnp.minimum(wid, extra)
        lo = c0 * _CW
        hi = lo + my_chunks * _CW
        is_last = wid == _NW - 1
        hi_eff = jnp.where(is_last, V, hi)

        def fetch(ci, buf):
            pltpu.async_copy(
                tabT_hbm.at[:, pl.ds((c0 + ci) * _CW, _CW)], buf, csem
            )

        @pl.when(my_chunks > 0)
        def _():
            fetch(jnp.int32(0), cka)

        if tail:
            @pl.when(is_last)
            def _():
                pltpu.async_copy(auxT_hbm, ckb.at[:, pl.ds(0, 128)], csem)

        pltpu.sync_copy(idx_hbm, idx_v)

        def scan_body(g, n):
            iv = idx_v[pl.ds(g * 16, 16)]
            bv = lax.iota(jnp.int32, 16) + g * 16
            ge = jnp.minimum(jnp.maximum(iv - lo + 1, 0), 1)
            lt = jnp.minimum(jnp.maximum(hi_eff - iv, 0), 1)
            m01 = ge * lt
            s = plsc.cumsum(m01)
            tgt = (n + s - m01) * m01 + B * (1 - m01)
            plsc.store_scatter(pos_v, [tgt], bv)
            plsc.store_scatter(row_v, [tgt], iv)
            return n + s[15]

        n_hits = lax.fori_loop(0, B // 16, scan_body, jnp.int32(0), unroll=4)
        n_groups = (n_hits + 15) // 16
        row_v[pl.ds(n_hits, 16)] = jnp.full((16,), V, jnp.int32)

        def drain_one():
            pltpu.make_async_copy(
                idx_hbm.at[pl.ds(0, D)], row_v.at[pl.ds(0, D)], wsem
            ).wait()

        def select_and_emit(buf, chunk_lo, width, cnt0):
            riota = lax.iota(jnp.int32, 16)

            def grp(g, cnt):
                rv = row_v[pl.ds(g * 16, 16)]
                ge = jnp.minimum(jnp.maximum(rv - chunk_lo + 1, 0), 1)
                lt = jnp.minimum(jnp.maximum(chunk_lo + width - rv, 0), 1)
                m01 = ge * lt
                s = plsc.cumsum(m01)
                tot = s[15]

                @pl.when(tot > 0)
                def _():
                    bv = pos_v[pl.ds(g * 16, 16)]
                    pfx = s - m01
                    for l in range(16):
                        @pl.when(m01[l] > 0)
                        def _(l=l):
                            c_l = cnt + pfx[l]

                            @pl.when(c_l >= 64)
                            def _():
                                drain_one()
                            col = jnp.full((16,), rv[l] - chunk_lo,
                                           jnp.int32)
                            slot = lax.rem(c_l, 64)
                            for cc in range(0, D, 16):
                                vals = plsc.load_gather(
                                    buf, [riota + cc, col]
                                )
                                ring_v[slot, pl.ds(cc, 16)] = vals
                            pltpu.async_copy(
                                ring_v.at[slot],
                                out_hbm.at[pl.ds(bv[l] * D, D)],
                                wsem,
                            )
                return cnt + tot

            return lax.fori_loop(0, n_groups, grp, cnt0)

        cnt0 = jnp.int32(0)
        if tail:
            def aux_path(c):
                pltpu.make_async_copy(
                    auxT_hbm, ckb.at[:, pl.ds(0, 128)], csem
                ).wait()
                return select_and_emit(ckb, jnp.int32(v_main), tail, c)

            cnt0 = lax.cond(is_last, aux_path, lambda c: c, cnt0)

        def body(ci, cnt):
            pltpu.make_async_copy(
                tabT_hbm.at[:, pl.ds(0, _CW)], cka, csem
            ).wait()
            even = lax.rem(ci, 2) == 0

            @pl.when(ci + 1 < my_chunks)
            def _():
                @pl.when(even)
                def _():
                    fetch(ci + 1, ckb)

                @pl.when(jnp.logical_not(even))
                def _():
                    fetch(ci + 1, cka)

            return lax.cond(
                even,
                lambda c: select_and_emit(cka, lo + ci * _CW, _CW, c),
                lambda c: select_and_emit(ckb, lo + ci * _CW, _CW, c),
                cnt,
            )

        total = lax.fori_loop(0, my_chunks, body, cnt0)

        def dr(i, _):
            drain_one()
            return ()

        lax.fori_loop(0, jnp.minimum(total, 64), dr, ())

    return k


def kernel(index, table):
    (B,) = index.shape
    V, D = table.shape
    assert B % 16 == 0 and D % 16 == 0
    idx = index.astype(jnp.int32)
    tabT = jnp.swapaxes(table, 0, 1)
    v_main = (V // _CW) * _CW
    tail = V - v_main
    auxT = jnp.swapaxes(
        jnp.pad(table[v_main:], ((0, 128 - tail), (0, 0))), 0, 1
    )
    out1d = _scan_call(B, D, V)(idx, tabT, auxT)
    return jnp.reshape(out1d, (B, D))
